# baseline (speedup 1.0000x reference)
import functools
import jax
import jax.numpy as jnp
from jax import lax
from jax.experimental import pallas as pl
from jax.experimental.pallas import tpu as pltpu
from jax.experimental.pallas import tpu_sc as plsc

N_FINE = 65536
N_COARSE = 8192
D_IN = 256
D_SKIP = 128
K = 3


_FB = 512
_SB = 32
_NCH = N_COARSE // 128


def _split3(a):
    h0 = a.astype(jnp.bfloat16)
    r1 = a - h0.astype(jnp.float32)
    h1 = r1.astype(jnp.bfloat16)
    r2 = r1 - h1.astype(jnp.float32)
    h2 = r2.astype(jnp.bfloat16)
    return h0, h1, h2


def _knn_body(ps_ref, postm2_ref, csq_ref,
              i0_ref, i1_ref, i2_ref, w0_ref, w1_ref, w2_ref,
              cross_ref, cand_ref, gidx_ref):
    ps = ps_ref[...]
    psq = jnp.sum(ps * ps, axis=1, keepdims=True)
    cross_ref[...] = jnp.dot(ps, postm2_ref[...],
                             preferred_element_type=jnp.float32,
                             precision=lax.Precision.HIGHEST)
    lane = lax.broadcasted_iota(jnp.int32, (_SB, 128), 1).astype(jnp.float32)
    big = jnp.float32(3.0e38)

    for sb in range(_FB // _SB):
        r0 = sb * _SB

        def chunk_body(j, carry):
            a1, a2, a3, i1, i2, i3 = carry
            v = (cross_ref[pl.ds(r0, _SB), pl.ds(j * 128, 128)]
                 + csq_ref[0:1, pl.ds(j * 128, 128)])
            jv = jnp.full((_SB, 128), j.astype(jnp.float32), jnp.float32)
            c1 = v < a1
            c2 = v < a2
            c3 = v < a3
            na1 = jnp.where(c1, v, a1)
            na2 = jnp.where(c1, a1, jnp.where(c2, v, a2))
            na3 = jnp.where(c2, a2, jnp.where(c3, v, a3))
            ni1 = jnp.where(c1, jv, i1)
            ni2 = jnp.where(c1, i1, jnp.where(c2, jv, i2))
            ni3 = jnp.where(c2, i2, jnp.where(c3, jv, i3))
            return (na1, na2, na3, ni1, ni2, ni3)

        fb = jnp.full((_SB, 128), big, jnp.float32)
        fz = jnp.zeros((_SB, 128), jnp.float32)
        a1, a2, a3, i1, i2, i3 = lax.fori_loop(
            0, _NCH, chunk_body, (fb, fb, fb, fz, fz, fz))
        cand_ref[pl.ds(r0, _SB), 0:128] = a1
        cand_ref[pl.ds(r0, _SB), 128:256] = a2
        cand_ref[pl.ds(r0, _SB), 256:384] = a3
        gidx_ref[pl.ds(r0, _SB), 0:128] = i1 * 128.0 + lane
        gidx_ref[pl.ds(r0, _SB), 128:256] = i2 * 128.0 + lane
        gidx_ref[pl.ds(r0, _SB), 256:384] = i3 * 128.0 + lane

    c = cand_ref[...]
    g = gidx_ref[...]

    def argmin_pass(d):
        m = jnp.min(d, axis=1, keepdims=True)
        s = jnp.min(jnp.where(d == m, g, big), axis=1, keepdims=True)
        return m, s

    m1, s1 = argmin_pass(c)
    c = jnp.where(g == s1, big, c)
    m2, s2 = argmin_pass(c)
    c = jnp.where(g == s2, big, c)
    m3, s3 = argmin_pass(c)

    w1 = 1.0 / jnp.maximum(m1 + psq, 1e-16)
    w2 = 1.0 / jnp.maximum(m2 + psq, 1e-16)
    w3 = 1.0 / jnp.maximum(m3 + psq, 1e-16)
    ws = w1 + w2 + w3
    i0_ref[...] = s1.astype(jnp.int32)
    i1_ref[...] = s2.astype(jnp.int32)
    i2_ref[...] = s3.astype(jnp.int32)
    w0_ref[...] = w1 / ws
    w1_ref[...] = w2 / ws
    w2_ref[...] = w3 / ws


def _knn(pos_skip, postm2, csq):
    n = pos_skip.shape[0]
    nblk = n // _FB
    col = pl.BlockSpec((_FB, 1), lambda i: (i, 0))
    return pl.pallas_call(
        _knn_body,
        grid=(nblk,),
        in_specs=[
            pl.BlockSpec((_FB, 3), lambda i: (i, 0)),
            pl.BlockSpec((3, N_COARSE), lambda i: (0, 0)),
            pl.BlockSpec((1, N_COARSE), lambda i: (0, 0)),
        ],
        out_specs=[col, col, col, col, col, col],
        out_shape=[
            jax.ShapeDtypeStruct((n, 1), jnp.int32),
            jax.ShapeDtypeStruct((n, 1), jnp.int32),
            jax.ShapeDtypeStruct((n, 1), jnp.int32),
            jax.ShapeDtypeStruct((n, 1), jnp.float32),
            jax.ShapeDtypeStruct((n, 1), jnp.float32),
            jax.ShapeDtypeStruct((n, 1), jnp.float32),
        ],
        scratch_shapes=[
            pltpu.VMEM((_FB, N_COARSE), jnp.float32),
            pltpu.VMEM((_FB, 3 * 128), jnp.float32),
            pltpu.VMEM((_FB, 3 * 128), jnp.float32),
        ],
    )(pos_skip, postm2, csq)


_NW = 32
_BPW = N_FINE // _NW
_C = 64
_NCHUNK = _BPW // _C


def _gc_body(bpw, nchunk, i0h, i1h, i2h, w0h, w1h, w2h, xh, yh,
             i0v, i1v, i2v, w0v, w1v, w2v, r0, r1, r2, yv,
             s0, s1, s2):
    wid = lax.axis_index("s") * 2 + lax.axis_index("c")
    wbase = wid * bpw

    def chunk(c, carry):
        base = wbase + c * _C
        pltpu.sync_copy(i0h.at[pl.ds(base, _C)], i0v)
        pltpu.sync_copy(i1h.at[pl.ds(base, _C)], i1v)
        pltpu.sync_copy(i2h.at[pl.ds(base, _C)], i2v)
        pltpu.sync_copy(w0h.at[pl.ds(base, _C)], w0v.at[pl.ds(0, _C)])
        pltpu.sync_copy(w1h.at[pl.ds(base, _C)], w1v.at[pl.ds(0, _C)])
        pltpu.sync_copy(w2h.at[pl.ds(base, _C)], w2v.at[pl.ds(0, _C)])
        cp0 = pltpu.async_copy(xh.at[i0v], r0, s0)
        cp1 = pltpu.async_copy(xh.at[i1v], r1, s1)
        cp2 = pltpu.async_copy(xh.at[i2v], r2, s2)
        cp0.wait()
        cp1.wait()
        cp2.wait()

        def row(i, carry2):
            a0 = jnp.full((16,), w0v[pl.ds(i, 16)][0], dtype=jnp.float32)
            a1 = jnp.full((16,), w1v[pl.ds(i, 16)][0], dtype=jnp.float32)
            a2 = jnp.full((16,), w2v[pl.ds(i, 16)][0], dtype=jnp.float32)
            for f in range(D_IN // 16):
                sl = pl.ds(f * 16, 16)
                yv[i, sl] = (a0 * r0[i, sl] + a1 * r1[i, sl]
                             + a2 * r2[i, sl])
            return carry2

        lax.fori_loop(0, _C, row, None)
        pltpu.sync_copy(yv, yh.at[pl.ds(base, _C)])
        return carry

    lax.fori_loop(0, nchunk, chunk, None)


def _gather_combine(i0, i1, i2, w0, w1, w2, x):
    n = i0.shape[0]
    bpw = n // _NW
    nchunk = bpw // _C
    mesh = plsc.VectorSubcoreMesh(core_axis_name="c", subcore_axis_name="s")
    fn = functools.partial(
        pl.kernel,
        mesh=mesh,
        out_type=jax.ShapeDtypeStruct((n, D_IN), jnp.float32),
        scratch_types=[
            pltpu.VMEM((_C,), jnp.int32),
            pltpu.VMEM((_C,), jnp.int32),
            pltpu.VMEM((_C,), jnp.int32),
            pltpu.VMEM((_C + 16,), jnp.float32),
            pltpu.VMEM((_C + 16,), jnp.float32),
            pltpu.VMEM((_C + 16,), jnp.float32),
            pltpu.VMEM((_C, D_IN), jnp.float32),
            pltpu.VMEM((_C, D_IN), jnp.float32),
            pltpu.VMEM((_C, D_IN), jnp.float32),
            pltpu.VMEM((_C, D_IN), jnp.float32),
            pltpu.SemaphoreType.DMA,
            pltpu.SemaphoreType.DMA,
            pltpu.SemaphoreType.DMA,
        ],
    )(functools.partial(_gc_body, bpw, nchunk))
    return fn(i0, i1, i2, w0, w1, w2, x)


_MB = 2048


def _dot3(a, b):
    a0 = a.astype(jnp.bfloat16)
    a1 = (a - a0.astype(jnp.float32)).astype(jnp.bfloat16)
    b0 = b.astype(jnp.bfloat16)
    b1 = (b - b0.astype(jnp.float32)).astype(jnp.bfloat16)
    lhs = jnp.concatenate([a0, a0, a1], axis=1)
    rhs = jnp.concatenate([b0, b1, b0], axis=0)
    return jnp.dot(lhs, rhs, preferred_element_type=jnp.float32)


def _mlp1_body(y_ref, xs_ref, w1a_ref, w1b_ref, b1_ref, h_ref, s_ref, q_ref):
    i = pl.program_id(0)
    h = _dot3(y_ref[...], w1a_ref[...])
    h = h + _dot3(xs_ref[...], w1b_ref[...])
    h = jnp.maximum(h + b1_ref[...], 0.0)
    h_ref[...] = h
    ps = jnp.sum(h, axis=0, keepdims=True)
    pq = jnp.sum(h * h, axis=0, keepdims=True)

    @pl.when(i == 0)
    def _():
        s_ref[...] = ps
        q_ref[...] = pq

    @pl.when(i != 0)
    def _():
        s_ref[...] = s_ref[...] + ps
        q_ref[...] = q_ref[...] + pq


def _mlp1(y, x_skip, W1a, W1b, b1):
    n = y.shape[0]
    nblk = n // _MB
    return pl.pallas_call(
        _mlp1_body,
        grid=(nblk,),
        in_specs=[
            pl.BlockSpec((_MB, D_IN), lambda i: (i, 0)),
            pl.BlockSpec((_MB, D_SKIP), lambda i: (i, 0)),
            pl.BlockSpec((D_IN, 256), lambda i: (0, 0)),
            pl.BlockSpec((D_SKIP, 256), lambda i: (0, 0)),
            pl.BlockSpec((1, 256), lambda i: (0, 0)),
        ],
        out_specs=[
            pl.BlockSpec((_MB, 256), lambda i: (i, 0)),
            pl.BlockSpec((1, 256), lambda i: (0, 0)),
            pl.BlockSpec((1, 256), lambda i: (0, 0)),
        ],
        out_shape=[
            jax.ShapeDtypeStruct((n, 256), jnp.float32),
            jax.ShapeDtypeStruct((1, 256), jnp.float32),
            jax.ShapeDtypeStruct((1, 256), jnp.float32),
        ],
    )(y, x_skip, W1a, W1b, b1)


def _mlp2_body(h_ref, s_ref, q_ref, g1_ref, be1_ref, w2_ref, b2_ref,
               h2_ref, s2_ref, q2_ref):
    i = pl.program_id(0)
    n = jnp.float32(N_FINE)
    m1 = s_ref[...] / n
    v1 = q_ref[...] / n - m1 * m1
    sc = g1_ref[...] / jnp.sqrt(v1 + 1e-5)
    w2_eff = w2_ref[...] * jnp.transpose(sc)
    c = jnp.dot(be1_ref[...] - m1 * sc, w2_ref[...],
                preferred_element_type=jnp.float32,
                precision=lax.Precision.HIGHEST) + b2_ref[...]
    h2 = _dot3(h_ref[...], w2_eff)
    h2 = jnp.maximum(h2 + c, 0.0)
    h2_ref[...] = h2
    ps = jnp.sum(h2, axis=0, keepdims=True)
    pq = jnp.sum(h2 * h2, axis=0, keepdims=True)

    @pl.when(i == 0)
    def _():
        s2_ref[...] = ps
        q2_ref[...] = pq

    @pl.when(i != 0)
    def _():
        s2_ref[...] = s2_ref[...] + ps
        q2_ref[...] = q2_ref[...] + pq


def _mlp2(h1, s1, q1, g1, be1, W2, b2):
    n = h1.shape[0]
    nblk = n // _MB
    return pl.pallas_call(
        _mlp2_body,
        grid=(nblk,),
        in_specs=[
            pl.BlockSpec((_MB, 256), lambda i: (i, 0)),
            pl.BlockSpec((1, 256), lambda i: (0, 0)),
            pl.BlockSpec((1, 256), lambda i: (0, 0)),
            pl.BlockSpec((1, 256), lambda i: (0, 0)),
            pl.BlockSpec((1, 256), lambda i: (0, 0)),
            pl.BlockSpec((256, 128), lambda i: (0, 0)),
            pl.BlockSpec((1, 128), lambda i: (0, 0)),
        ],
        out_specs=[
            pl.BlockSpec((_MB, 128), lambda i: (i, 0)),
            pl.BlockSpec((1, 128), lambda i: (0, 0)),
            pl.BlockSpec((1, 128), lambda i: (0, 0)),
        ],
        out_shape=[
            jax.ShapeDtypeStruct((n, 128), jnp.float32),
            jax.ShapeDtypeStruct((1, 128), jnp.float32),
            jax.ShapeDtypeStruct((1, 128), jnp.float32),
        ],
    )(h1, s1, q1, g1, be1, W2, b2)


def _fin_body(h2_ref, s_ref, q_ref, g2_ref, be2_ref, o_ref):
    n = jnp.float32(N_FINE)
    m = s_ref[...] / n
    v = q_ref[...] / n - m * m
    sc = g2_ref[...] / jnp.sqrt(v + 1e-5)
    o_ref[...] = (h2_ref[...] - m) * sc + be2_ref[...]


def _fin(h2, s2, q2, g2, be2):
    n = h2.shape[0]
    nblk = n // _MB
    return pl.pallas_call(
        _fin_body,
        grid=(nblk,),
        in_specs=[
            pl.BlockSpec((_MB, 128), lambda i: (i, 0)),
            pl.BlockSpec((1, 128), lambda i: (0, 0)),
            pl.BlockSpec((1, 128), lambda i: (0, 0)),
            pl.BlockSpec((1, 128), lambda i: (0, 0)),
            pl.BlockSpec((1, 128), lambda i: (0, 0)),
        ],
        out_specs=pl.BlockSpec((_MB, 128), lambda i: (i, 0)),
        out_shape=jax.ShapeDtypeStruct((n, 128), jnp.float32),
    )(h2, s2, q2, g2, be2)


@jax.jit
def _run(x, pos, x_skip, pos_skip, W1, b1, g1, be1, W2, b2, g2, be2):
    postm2 = -2.0 * jnp.transpose(pos)
    csq = jnp.sum(pos * pos, axis=1)[None, :]
    W1a = W1[:D_IN]
    W1b = W1[D_IN:]
    H = N_FINE // 2

    knn0 = _knn(pos_skip[:H], postm2, csq)
    knn1 = _knn(pos_skip[H:], postm2, csq)
    y0 = _gather_combine(*[a.reshape(H) for a in knn0], x)
    y1 = _gather_combine(*[a.reshape(H) for a in knn1], x)
    h1_0, s1a, q1a = _mlp1(y0, x_skip[:H], W1a, W1b, b1.reshape(1, -1))
    h1_1, s1b, q1b = _mlp1(y1, x_skip[H:], W1a, W1b, b1.reshape(1, -1))
    s1 = s1a + s1b
    q1 = q1a + q1b
    h2_0, s2a, q2a = _mlp2(h1_0, s1, q1, g1.reshape(1, -1),
                           be1.reshape(1, -1), W2, b2.reshape(1, -1))
    h2_1, s2b, q2b = _mlp2(h1_1, s1, q1, g1.reshape(1, -1),
                           be1.reshape(1, -1), W2, b2.reshape(1, -1))
    s2 = s2a + s2b
    q2 = q2a + q2b
    out0 = _fin(h2_0, s2, q2, g2.reshape(1, -1), be2.reshape(1, -1))
    out1 = _fin(h2_1, s2, q2, g2.reshape(1, -1), be2.reshape(1, -1))
    return jnp.concatenate([out0, out1], axis=0)


def kernel(x, pos, batch, x_skip, pos_skip, batch_skip, W1, b1, g1, be1,
           W2, b2, g2, be2):
    out = _run(x, pos, x_skip, pos_skip, W1, b1, g1, be1, W2, b2, g2, be2)
    return (out, pos_skip, batch_skip)


# speedup vs baseline: 1.4613x; 1.4613x over previous
import functools
import jax
import jax.numpy as jnp
from jax import lax
from jax.experimental import pallas as pl
from jax.experimental.pallas import tpu as pltpu
from jax.experimental.pallas import tpu_sc as plsc

N_FINE = 65536
N_COARSE = 8192
D_IN = 256
D_SKIP = 128
K = 3


_FB = 512
_SB = 32
_NCH = N_COARSE // 128


def _split3(a):
    h0 = a.astype(jnp.bfloat16)
    r1 = a - h0.astype(jnp.float32)
    h1 = r1.astype(jnp.bfloat16)
    r2 = r1 - h1.astype(jnp.float32)
    h2 = r2.astype(jnp.bfloat16)
    return h0, h1, h2


def _knn_body(ps_ref, lhs_ref, postm2_ref, csq_ref,
              i0_ref, i1_ref, i2_ref, w0_ref, w1_ref, w2_ref,
              cross_ref, cand_ref, gidx_ref):
    ps = ps_ref[...]
    psq = jnp.sum(ps * ps, axis=1, keepdims=True)
    cross_ref[...] = jnp.dot(lhs_ref[...], postm2_ref[...],
                             preferred_element_type=jnp.float32)
    lane = lax.broadcasted_iota(jnp.int32, (_SB, 128), 1).astype(jnp.float32)
    big = jnp.float32(3.0e38)

    for sb in range(_FB // _SB):
        r0 = sb * _SB

        def chunk_body(j, carry):
            a1, a2, a3, i1, i2, i3 = carry
            v = (cross_ref[pl.ds(r0, _SB), pl.ds(j * 128, 128)]
                 + csq_ref[0:1, pl.ds(j * 128, 128)])
            jv = jnp.full((_SB, 128), j.astype(jnp.float32), jnp.float32)
            c1 = v < a1
            c2 = v < a2
            c3 = v < a3
            na1 = jnp.where(c1, v, a1)
            na2 = jnp.where(c1, a1, jnp.where(c2, v, a2))
            na3 = jnp.where(c2, a2, jnp.where(c3, v, a3))
            ni1 = jnp.where(c1, jv, i1)
            ni2 = jnp.where(c1, i1, jnp.where(c2, jv, i2))
            ni3 = jnp.where(c2, i2, jnp.where(c3, jv, i3))
            return (na1, na2, na3, ni1, ni2, ni3)

        fb = jnp.full((_SB, 128), big, jnp.float32)
        fz = jnp.zeros((_SB, 128), jnp.float32)
        a1, a2, a3, i1, i2, i3 = lax.fori_loop(
            0, _NCH, chunk_body, (fb, fb, fb, fz, fz, fz))
        cand_ref[pl.ds(r0, _SB), 0:128] = a1
        cand_ref[pl.ds(r0, _SB), 128:256] = a2
        cand_ref[pl.ds(r0, _SB), 256:384] = a3
        gidx_ref[pl.ds(r0, _SB), 0:128] = i1 * 128.0 + lane
        gidx_ref[pl.ds(r0, _SB), 128:256] = i2 * 128.0 + lane
        gidx_ref[pl.ds(r0, _SB), 256:384] = i3 * 128.0 + lane

    c = cand_ref[...]
    g = gidx_ref[...]

    def argmin_pass(d):
        m = jnp.min(d, axis=1, keepdims=True)
        s = jnp.min(jnp.where(d == m, g, big), axis=1, keepdims=True)
        return m, s

    m1, s1 = argmin_pass(c)
    c = jnp.where(g == s1, big, c)
    m2, s2 = argmin_pass(c)
    c = jnp.where(g == s2, big, c)
    m3, s3 = argmin_pass(c)

    w1 = 1.0 / jnp.maximum(m1 + psq, 1e-16)
    w2 = 1.0 / jnp.maximum(m2 + psq, 1e-16)
    w3 = 1.0 / jnp.maximum(m3 + psq, 1e-16)
    ws = w1 + w2 + w3
    i0_ref[...] = s1.astype(jnp.int32)
    i1_ref[...] = s2.astype(jnp.int32)
    i2_ref[...] = s3.astype(jnp.int32)
    w0_ref[...] = w1 / ws
    w1_ref[...] = w2 / ws
    w2_ref[...] = w3 / ws


def _knn(pos_skip, lhs18, postm2, csq):
    n = pos_skip.shape[0]
    nblk = n // _FB
    col = pl.BlockSpec((_FB, 1), lambda i: (i, 0))
    return pl.pallas_call(
        _knn_body,
        grid=(nblk,),
        in_specs=[
            pl.BlockSpec((_FB, 3), lambda i: (i, 0)),
            pl.BlockSpec((_FB, 18), lambda i: (i, 0)),
            pl.BlockSpec((18, N_COARSE), lambda i: (0, 0)),
            pl.BlockSpec((1, N_COARSE), lambda i: (0, 0)),
        ],
        out_specs=[col, col, col, col, col, col],
        out_shape=[
            jax.ShapeDtypeStruct((n, 1), jnp.int32),
            jax.ShapeDtypeStruct((n, 1), jnp.int32),
            jax.ShapeDtypeStruct((n, 1), jnp.int32),
            jax.ShapeDtypeStruct((n, 1), jnp.float32),
            jax.ShapeDtypeStruct((n, 1), jnp.float32),
            jax.ShapeDtypeStruct((n, 1), jnp.float32),
        ],
        scratch_shapes=[
            pltpu.VMEM((_FB, N_COARSE), jnp.float32),
            pltpu.VMEM((_FB, 3 * 128), jnp.float32),
            pltpu.VMEM((_FB, 3 * 128), jnp.float32),
        ],
    )(pos_skip, lhs18, postm2, csq)


_NW = 32
_BPW = N_FINE // _NW
_C = 64
_NCHUNK = _BPW // _C


def _gc_body(bpw, nchunk, i0h, i1h, i2h, w0h, w1h, w2h, xh, yh,
             i0v, i1v, i2v, w0v, w1v, w2v, r0, r1, r2, yv,
             s0, s1, s2):
    wid = lax.axis_index("s") * 2 + lax.axis_index("c")
    wbase = wid * bpw

    def chunk(c, carry):
        base = wbase + c * _C
        pltpu.sync_copy(i0h.at[pl.ds(base, _C)], i0v)
        pltpu.sync_copy(i1h.at[pl.ds(base, _C)], i1v)
        pltpu.sync_copy(i2h.at[pl.ds(base, _C)], i2v)
        pltpu.sync_copy(w0h.at[pl.ds(base, _C)], w0v.at[pl.ds(0, _C)])
        pltpu.sync_copy(w1h.at[pl.ds(base, _C)], w1v.at[pl.ds(0, _C)])
        pltpu.sync_copy(w2h.at[pl.ds(base, _C)], w2v.at[pl.ds(0, _C)])
        cp0 = pltpu.async_copy(xh.at[i0v], r0, s0)
        cp1 = pltpu.async_copy(xh.at[i1v], r1, s1)
        cp2 = pltpu.async_copy(xh.at[i2v], r2, s2)
        cp0.wait()
        cp1.wait()
        cp2.wait()

        def row(i, carry2):
            a0 = jnp.full((16,), w0v[pl.ds(i, 16)][0], dtype=jnp.float32)
            a1 = jnp.full((16,), w1v[pl.ds(i, 16)][0], dtype=jnp.float32)
            a2 = jnp.full((16,), w2v[pl.ds(i, 16)][0], dtype=jnp.float32)
            for f in range(D_IN // 16):
                sl = pl.ds(f * 16, 16)
                yv[i, sl] = (a0 * r0[i, sl] + a1 * r1[i, sl]
                             + a2 * r2[i, sl])
            return carry2

        lax.fori_loop(0, _C, row, None)
        pltpu.sync_copy(yv, yh.at[pl.ds(base, _C)])
        return carry

    lax.fori_loop(0, nchunk, chunk, None)


def _gather_combine(i0, i1, i2, w0, w1, w2, x):
    n = i0.shape[0]
    bpw = n // _NW
    nchunk = bpw // _C
    mesh = plsc.VectorSubcoreMesh(core_axis_name="c", subcore_axis_name="s")
    fn = functools.partial(
        pl.kernel,
        mesh=mesh,
        out_type=jax.ShapeDtypeStruct((n, D_IN), jnp.float32),
        scratch_types=[
            pltpu.VMEM((_C,), jnp.int32),
            pltpu.VMEM((_C,), jnp.int32),
            pltpu.VMEM((_C,), jnp.int32),
            pltpu.VMEM((_C + 16,), jnp.float32),
            pltpu.VMEM((_C + 16,), jnp.float32),
            pltpu.VMEM((_C + 16,), jnp.float32),
            pltpu.VMEM((_C, D_IN), jnp.float32),
            pltpu.VMEM((_C, D_IN), jnp.float32),
            pltpu.VMEM((_C, D_IN), jnp.float32),
            pltpu.VMEM((_C, D_IN), jnp.float32),
            pltpu.SemaphoreType.DMA,
            pltpu.SemaphoreType.DMA,
            pltpu.SemaphoreType.DMA,
        ],
    )(functools.partial(_gc_body, bpw, nchunk))
    return fn(i0, i1, i2, w0, w1, w2, x)


_MB = 2048


def _dot3(a, b):
    a0 = a.astype(jnp.bfloat16)
    a1 = (a - a0.astype(jnp.float32)).astype(jnp.bfloat16)
    b0 = b.astype(jnp.bfloat16)
    b1 = (b - b0.astype(jnp.float32)).astype(jnp.bfloat16)
    lhs = jnp.concatenate([a0, a0, a1], axis=1)
    rhs = jnp.concatenate([b0, b1, b0], axis=0)
    return jnp.dot(lhs, rhs, preferred_element_type=jnp.float32)


def _mlp1_body(y_ref, xs_ref, w1a_ref, w1b_ref, b1_ref, h_ref, s_ref, q_ref):
    i = pl.program_id(0)
    h = _dot3(y_ref[...], w1a_ref[...])
    h = h + _dot3(xs_ref[...], w1b_ref[...])
    h = jnp.maximum(h + b1_ref[...], 0.0)
    h_ref[...] = h
    ps = jnp.sum(h, axis=0, keepdims=True)
    pq = jnp.sum(h * h, axis=0, keepdims=True)

    @pl.when(i == 0)
    def _():
        s_ref[...] = ps
        q_ref[...] = pq

    @pl.when(i != 0)
    def _():
        s_ref[...] = s_ref[...] + ps
        q_ref[...] = q_ref[...] + pq


def _mlp1(y, x_skip, W1a, W1b, b1):
    n = y.shape[0]
    nblk = n // _MB
    return pl.pallas_call(
        _mlp1_body,
        grid=(nblk,),
        in_specs=[
            pl.BlockSpec((_MB, D_IN), lambda i: (i, 0)),
            pl.BlockSpec((_MB, D_SKIP), lambda i: (i, 0)),
            pl.BlockSpec((D_IN, 256), lambda i: (0, 0)),
            pl.BlockSpec((D_SKIP, 256), lambda i: (0, 0)),
            pl.BlockSpec((1, 256), lambda i: (0, 0)),
        ],
        out_specs=[
            pl.BlockSpec((_MB, 256), lambda i: (i, 0)),
            pl.BlockSpec((1, 256), lambda i: (0, 0)),
            pl.BlockSpec((1, 256), lambda i: (0, 0)),
        ],
        out_shape=[
            jax.ShapeDtypeStruct((n, 256), jnp.float32),
            jax.ShapeDtypeStruct((1, 256), jnp.float32),
            jax.ShapeDtypeStruct((1, 256), jnp.float32),
        ],
    )(y, x_skip, W1a, W1b, b1)


def _mlp2_body(h_ref, s_ref, q_ref, g1_ref, be1_ref, w2_ref, b2_ref,
               h2_ref, s2_ref, q2_ref):
    i = pl.program_id(0)
    n = jnp.float32(N_FINE)
    m1 = s_ref[...] / n
    v1 = q_ref[...] / n - m1 * m1
    sc = g1_ref[...] / jnp.sqrt(v1 + 1e-5)
    w2_eff = w2_ref[...] * jnp.transpose(sc)
    c = jnp.dot(be1_ref[...] - m1 * sc, w2_ref[...],
                preferred_element_type=jnp.float32,
                precision=lax.Precision.HIGHEST) + b2_ref[...]
    h2 = _dot3(h_ref[...], w2_eff)
    h2 = jnp.maximum(h2 + c, 0.0)
    h2_ref[...] = h2
    ps = jnp.sum(h2, axis=0, keepdims=True)
    pq = jnp.sum(h2 * h2, axis=0, keepdims=True)

    @pl.when(i == 0)
    def _():
        s2_ref[...] = ps
        q2_ref[...] = pq

    @pl.when(i != 0)
    def _():
        s2_ref[...] = s2_ref[...] + ps
        q2_ref[...] = q2_ref[...] + pq


def _mlp2(h1, s1, q1, g1, be1, W2, b2):
    n = h1.shape[0]
    nblk = n // _MB
    return pl.pallas_call(
        _mlp2_body,
        grid=(nblk,),
        in_specs=[
            pl.BlockSpec((_MB, 256), lambda i: (i, 0)),
            pl.BlockSpec((1, 256), lambda i: (0, 0)),
            pl.BlockSpec((1, 256), lambda i: (0, 0)),
            pl.BlockSpec((1, 256), lambda i: (0, 0)),
            pl.BlockSpec((1, 256), lambda i: (0, 0)),
            pl.BlockSpec((256, 128), lambda i: (0, 0)),
            pl.BlockSpec((1, 128), lambda i: (0, 0)),
        ],
        out_specs=[
            pl.BlockSpec((_MB, 128), lambda i: (i, 0)),
            pl.BlockSpec((1, 128), lambda i: (0, 0)),
            pl.BlockSpec((1, 128), lambda i: (0, 0)),
        ],
        out_shape=[
            jax.ShapeDtypeStruct((n, 128), jnp.float32),
            jax.ShapeDtypeStruct((1, 128), jnp.float32),
            jax.ShapeDtypeStruct((1, 128), jnp.float32),
        ],
    )(h1, s1, q1, g1, be1, W2, b2)


def _fin_body(h2_ref, s_ref, q_ref, g2_ref, be2_ref, o_ref):
    n = jnp.float32(N_FINE)
    m = s_ref[...] / n
    v = q_ref[...] / n - m * m
    sc = g2_ref[...] / jnp.sqrt(v + 1e-5)
    o_ref[...] = (h2_ref[...] - m) * sc + be2_ref[...]


def _fin(h2, s2, q2, g2, be2):
    n = h2.shape[0]
    nblk = n // _MB
    return pl.pallas_call(
        _fin_body,
        grid=(nblk,),
        in_specs=[
            pl.BlockSpec((_MB, 128), lambda i: (i, 0)),
            pl.BlockSpec((1, 128), lambda i: (0, 0)),
            pl.BlockSpec((1, 128), lambda i: (0, 0)),
            pl.BlockSpec((1, 128), lambda i: (0, 0)),
            pl.BlockSpec((1, 128), lambda i: (0, 0)),
        ],
        out_specs=pl.BlockSpec((_MB, 128), lambda i: (i, 0)),
        out_shape=jax.ShapeDtypeStruct((n, 128), jnp.float32),
    )(h2, s2, q2, g2, be2)


@jax.jit
def _run(x, pos, x_skip, pos_skip, W1, b1, g1, be1, W2, b2, g2, be2):
    pm = -2.0 * jnp.transpose(pos)
    pb0, pb1, pb2 = _split3(pm)
    postm2 = jnp.concatenate([pb0, pb1, pb0, pb2, pb0, pb1], axis=0)
    a0, a1, a2 = _split3(pos_skip)
    lhs18 = jnp.concatenate([a0, a0, a1, a0, a2, a1], axis=1)
    csq = jnp.sum(pos * pos, axis=1)[None, :]
    W1a = W1[:D_IN]
    W1b = W1[D_IN:]
    H = N_FINE // 2

    knn0 = _knn(pos_skip[:H], lhs18[:H], postm2, csq)
    knn1 = _knn(pos_skip[H:], lhs18[H:], postm2, csq)
    y0 = _gather_combine(*[a.reshape(H) for a in knn0], x)
    y1 = _gather_combine(*[a.reshape(H) for a in knn1], x)
    h1_0, s1a, q1a = _mlp1(y0, x_skip[:H], W1a, W1b, b1.reshape(1, -1))
    h1_1, s1b, q1b = _mlp1(y1, x_skip[H:], W1a, W1b, b1.reshape(1, -1))
    s1 = s1a + s1b
    q1 = q1a + q1b
    h2_0, s2a, q2a = _mlp2(h1_0, s1, q1, g1.reshape(1, -1),
                           be1.reshape(1, -1), W2, b2.reshape(1, -1))
    h2_1, s2b, q2b = _mlp2(h1_1, s1, q1, g1.reshape(1, -1),
                           be1.reshape(1, -1), W2, b2.reshape(1, -1))
    s2 = s2a + s2b
    q2 = q2a + q2b
    out0 = _fin(h2_0, s2, q2, g2.reshape(1, -1), be2.reshape(1, -1))
    out1 = _fin(h2_1, s2, q2, g2.reshape(1, -1), be2.reshape(1, -1))
    return jnp.concatenate([out0, out1], axis=0)


def kernel(x, pos, batch, x_skip, pos_skip, batch_skip, W1, b1, g1, be1,
           W2, b2, g2, be2):
    out = _run(x, pos, x_skip, pos_skip, W1, b1, g1, be1, W2, b2, g2, be2)
    return (out, pos_skip, batch_skip)
